# Initial kernel scaffold; baseline (speedup 1.0000x reference)
import jax
import jax.numpy as jnp
from jax.experimental import pallas as pl


def kernel(x, edge_index, edge_attr, batch, params):
    raise NotImplementedError("write your pallas kernel here")


# speedup vs baseline: 15.2788x; 15.2788x over previous
from functools import partial

import jax
import jax.numpy as jnp
import numpy as np
from jax.experimental import pallas as pl
from jax.experimental.pallas import tpu as pltpu

N = 10000
E = 160000
G = 512
T = 5
F = 75
TF = 375
FO = 15
L = 4
HID = 75
Fp = 128
TFp = 384
BN_ROWS = 1000
NT = N // BN_ROWS
DEG_HIST = [0, 0, 0, 0, 0, 0, 0, 0, 100, 300, 600, 900, 1200, 1400, 1500,
            1400, 1200, 900, 600, 300, 100]
_dh = np.asarray(DEG_HIST, np.float64)
AVG_LOG = float((np.log(np.arange(_dh.shape[0]) + 1.0) * _dh).sum() / _dh.sum())


def _pad2(a, r, c):
    return jnp.pad(a, ((0, r - a.shape[0]), (0, c - a.shape[1])))


def _bn_relu(y, stats):
    m, rstd, g, bb = (stats[0:1, :], stats[1:2, :], stats[2:3, :],
                      stats[3:4, :])
    return jnp.maximum((y - m) * rstd * g + bb, 0.0)


def _ab_body(h_ref, stats_ref, wd_ref, ws_ref, a_ref, b_ref, *, apply_bn):
    h = h_ref[...]
    if apply_bn:
        h = _bn_relu(h, stats_ref[...])
    a_ref[...] = jnp.dot(h, wd_ref[...], preferred_element_type=jnp.float32)
    b_ref[...] = jnp.dot(h, ws_ref[...], preferred_element_type=jnp.float32)


def _k_ab(h, stats, wd, ws, apply_bn):
    return pl.pallas_call(
        partial(_ab_body, apply_bn=apply_bn),
        grid=(NT,),
        in_specs=[
            pl.BlockSpec((BN_ROWS, Fp), lambda i: (i, 0)),
            pl.BlockSpec((4, Fp), lambda i: (0, 0)),
            pl.BlockSpec((Fp, TFp), lambda i: (0, 0)),
            pl.BlockSpec((Fp, TFp), lambda i: (0, 0)),
        ],
        out_specs=[
            pl.BlockSpec((BN_ROWS, TFp), lambda i: (i, 0)),
            pl.BlockSpec((BN_ROWS, TFp), lambda i: (i, 0)),
        ],
        out_shape=[
            jax.ShapeDtypeStruct((N, TFp), jnp.float32),
            jax.ShapeDtypeStruct((N, TFp), jnp.float32),
        ],
    )(h, stats, wd, ws)


def _post_body(ns_ref, a_ref, s1_ref, s2_ref, mn_ref, mx_ref, h_ref,
               stats_in_ref, wx_ref, w1_ref, w2_ref, w3_ref, lw_ref,
               bias_ref, y_ref, stats_ref, acc_ref, *, apply_bn):
    i = pl.program_id(0)
    deg = ns_ref[:, 0:1]
    inv = ns_ref[:, 1:2]
    sc1 = ns_ref[:, 2:3]
    sc2 = ns_ref[:, 3:4]
    has = ns_ref[:, 4:5]
    A = a_ref[...]
    S1 = s1_ref[...]
    S2 = s2_ref[...]
    mean = (deg * A + S1) * inv
    sq = (deg * A * A + 2.0 * A * S1 + S2) * inv
    var = jnp.maximum(sq - mean * mean, 0.0)
    std = jnp.sqrt(var + 1e-5)
    mn = has * (A + mn_ref[...])
    mx = has * (A + mx_ref[...])

    h = h_ref[...]
    if apply_bn:
        h = _bn_relu(h, stats_in_ref[...])
    f32 = jnp.float32
    y = jnp.dot(h, wx_ref[...], preferred_element_type=f32)
    for k, agg in enumerate((mean, mn, mx, std)):
        y0 = jnp.dot(agg, w1_ref[k], preferred_element_type=f32)
        y1 = jnp.dot(agg, w2_ref[k], preferred_element_type=f32)
        y2 = jnp.dot(agg, w3_ref[k], preferred_element_type=f32)
        y = y + y0 + sc1 * y1 + sc2 * y2
    y = y + bias_ref[0:1, :]
    y = jnp.dot(y, lw_ref[...], preferred_element_type=f32) + bias_ref[1:2, :]
    y_ref[...] = y

    @pl.when(i == 0)
    def _():
        acc_ref[...] = jnp.zeros_like(acc_ref)

    acc_ref[0:1, :] += jnp.sum(y, axis=0, keepdims=True)
    acc_ref[1:2, :] += jnp.sum(y * y, axis=0, keepdims=True)

    @pl.when(i == NT - 1)
    def _():
        m = acc_ref[0:1, :] / float(N)
        v = acc_ref[1:2, :] / float(N) - m * m
        rstd = jax.lax.rsqrt(v + 1e-5)
        stats_ref[...] = jnp.concatenate(
            [m, rstd, bias_ref[2:3, :], bias_ref[3:4, :]], axis=0)


def _k_post(ns, A, S1, S2, MN, MX, h, stats_in, wx, w123, lw, bias, apply_bn):
    w1, w2, w3 = w123
    return pl.pallas_call(
        partial(_post_body, apply_bn=apply_bn),
        grid=(NT,),
        in_specs=[
            pl.BlockSpec((BN_ROWS, 128), lambda i: (i, 0)),
            pl.BlockSpec((BN_ROWS, TFp), lambda i: (i, 0)),
            pl.BlockSpec((BN_ROWS, TFp), lambda i: (i, 0)),
            pl.BlockSpec((BN_ROWS, TFp), lambda i: (i, 0)),
            pl.BlockSpec((BN_ROWS, TFp), lambda i: (i, 0)),
            pl.BlockSpec((BN_ROWS, TFp), lambda i: (i, 0)),
            pl.BlockSpec((BN_ROWS, Fp), lambda i: (i, 0)),
            pl.BlockSpec((4, Fp), lambda i: (0, 0)),
            pl.BlockSpec((Fp, Fp), lambda i: (0, 0)),
            pl.BlockSpec((4, TFp, Fp), lambda i: (0, 0, 0)),
            pl.BlockSpec((4, TFp, Fp), lambda i: (0, 0, 0)),
            pl.BlockSpec((4, TFp, Fp), lambda i: (0, 0, 0)),
            pl.BlockSpec((Fp, Fp), lambda i: (0, 0)),
            pl.BlockSpec((4, Fp), lambda i: (0, 0)),
        ],
        out_specs=[
            pl.BlockSpec((BN_ROWS, Fp), lambda i: (i, 0)),
            pl.BlockSpec((4, Fp), lambda i: (0, 0)),
        ],
        out_shape=[
            jax.ShapeDtypeStruct((N, Fp), jnp.float32),
            jax.ShapeDtypeStruct((4, Fp), jnp.float32),
        ],
        scratch_shapes=[pltpu.VMEM((8, Fp), jnp.float32)],
    )(ns, A, S1, S2, MN, MX, h, stats_in, wx, w1, w2, w3, lw, bias)


def _pool_body(y_ref, stats_ref, ns_ref, w1_ref, b1_ref, w2_ref, b2_ref,
               w3_ref, b3_ref, out_ref, acc_ref):
    i = pl.program_id(0)

    @pl.when(i == 0)
    def _():
        acc_ref[...] = jnp.zeros_like(acc_ref)

    h = _bn_relu(y_ref[...], stats_ref[...])
    bid = ns_ref[:, 5].astype(jnp.int32)
    gids = jax.lax.broadcasted_iota(jnp.int32, (G, BN_ROWS), 0)
    onehot = jnp.where(gids == bid[None, :], 1.0, 0.0)
    acc_ref[...] += jnp.dot(onehot, h, preferred_element_type=jnp.float32)

    @pl.when(i == NT - 1)
    def _():
        f32 = jnp.float32
        z = acc_ref[...]
        z = jnp.maximum(jnp.dot(z, w1_ref[...], preferred_element_type=f32)
                        + b1_ref[0:1, :], 0.0)
        z = jnp.maximum(jnp.dot(z, w2_ref[...], preferred_element_type=f32)
                        + b2_ref[0:1, :], 0.0)
        z = jnp.dot(z, w3_ref[...], preferred_element_type=f32) + b3_ref[0:1, :]
        out_ref[...] = 1.0 / (1.0 + jnp.exp(-z))


def _k_pool(y, stats, ns, w1, b1, w2, b2, w3, b3):
    return pl.pallas_call(
        _pool_body,
        grid=(NT,),
        in_specs=[
            pl.BlockSpec((BN_ROWS, Fp), lambda i: (i, 0)),
            pl.BlockSpec((4, Fp), lambda i: (0, 0)),
            pl.BlockSpec((BN_ROWS, 128), lambda i: (i, 0)),
            pl.BlockSpec((Fp, 128), lambda i: (0, 0)),
            pl.BlockSpec((1, 128), lambda i: (0, 0)),
            pl.BlockSpec((128, 128), lambda i: (0, 0)),
            pl.BlockSpec((1, 128), lambda i: (0, 0)),
            pl.BlockSpec((128, 128), lambda i: (0, 0)),
            pl.BlockSpec((1, 128), lambda i: (0, 0)),
        ],
        out_specs=pl.BlockSpec((G, 128), lambda i: (0, 0)),
        out_shape=jax.ShapeDtypeStruct((G, 128), jnp.float32),
        scratch_shapes=[pltpu.VMEM((G, Fp), jnp.float32)],
    )(y, stats, ns, w1, b1, w2, b2, w3, b3)


def _prep_params(params):
    p = {}
    for i in range(L):
        w = params["pre_W"][i]
        wd = w[:, :F].transpose(1, 0, 2).reshape(F, TF)
        ws = w[:, F:2 * F].transpose(1, 0, 2).reshape(F, TF)
        p[f"wd{i}"] = _pad2(wd, Fp, TFp)
        p[f"ws{i}"] = _pad2(ws, Fp, TFp)
        etab = params["edge_emb"] @ params["enc_W"][i] + params["enc_b"][i]
        ET = (jnp.einsum('af,tfg->atg', etab, w[:, 2 * F:])
              + params["pre_b"][i][None]).reshape(4, TF)
        p[f"et{i}"] = _pad2(ET, 8, TFp)
        pw = params["post_W"][i]
        wx = jnp.zeros((Fp, Fp), jnp.float32)
        for t in range(T):
            wx = wx.at[:F, t * FO:(t + 1) * FO].set(pw[t, :F])
        p[f"wx{i}"] = wx
        for s in range(3):
            blocks = []
            for k in range(4):
                wkt = jnp.zeros((TFp, Fp), jnp.float32)
                for t in range(T):
                    rows = pw[t, F + (4 * s + k) * F: F + (4 * s + k + 1) * F]
                    wkt = wkt.at[t * F:(t + 1) * F,
                                 t * FO:(t + 1) * FO].set(rows)
                blocks.append(wkt)
            p[f"w{s + 1}_{i}"] = jnp.stack(blocks)
        p[f"lw{i}"] = _pad2(params["lin_W"][i], Fp, Fp)
        bias = jnp.zeros((4, Fp), jnp.float32)
        bias = bias.at[0, :T * FO].set(params["post_b"][i].reshape(-1))
        bias = bias.at[1, :HID].set(params["lin_b"][i])
        bias = bias.at[2, :HID].set(params["bn_g"][i])
        bias = bias.at[3, :HID].set(params["bn_b"][i])
        p[f"bias{i}"] = bias
    p["mlp_W1"] = _pad2(params["mlp_W1"], Fp, 128)
    p["mlp_b1"] = _pad2(params["mlp_b1"][None, :], 1, 128)
    p["mlp_W2"] = _pad2(params["mlp_W2"], 128, 128)
    p["mlp_b2"] = _pad2(params["mlp_b2"][None, :], 1, 128)
    p["mlp_W3"] = _pad2(params["mlp_W3"], 128, 128)
    p["mlp_b3"] = _pad2(params["mlp_b3"][None, :], 1, 128)
    return p


def kernel(x, edge_index, edge_attr, batch, params):
    src = edge_index[0]
    dst = edge_index[1]
    perm = jnp.argsort(dst)
    dst_s = dst[perm]
    src_s = src[perm]
    attr_s = edge_attr[perm]
    rowptr = jnp.searchsorted(
        dst_s, jnp.arange(N + 1, dtype=jnp.int32)).astype(jnp.int32)
    deg = (rowptr[1:] - rowptr[:-1]).astype(jnp.float32)
    degc = jnp.maximum(deg, 1.0)
    ld = jnp.log(degc + 1.0)
    ns = jnp.zeros((N, 128), jnp.float32)
    ns = ns.at[:, 0].set(deg)
    ns = ns.at[:, 1].set(1.0 / degc)
    ns = ns.at[:, 2].set(ld / AVG_LOG)
    ns = ns.at[:, 3].set(AVG_LOG / ld)
    ns = ns.at[:, 4].set((deg > 0).astype(jnp.float32))
    ns = ns.at[:, 5].set(batch.astype(jnp.float32))

    p = _prep_params(params)
    h = _pad2(params["node_emb"], 32, Fp)[x]
    stats = jnp.zeros((4, Fp), jnp.float32)
    y = h

    for i in range(L):
        A, B = _k_ab(y, stats, p[f"wd{i}"], p[f"ws{i}"], apply_bn=(i > 0))
        c = B[src_s] + p[f"et{i}"][attr_s]
        S1 = jax.ops.segment_sum(c, dst_s, num_segments=N,
                                 indices_are_sorted=True)
        S2 = jax.ops.segment_sum(c * c, dst_s, num_segments=N,
                                 indices_are_sorted=True)
        MN = jax.ops.segment_min(c, dst_s, num_segments=N,
                                 indices_are_sorted=True)
        MX = jax.ops.segment_max(c, dst_s, num_segments=N,
                                 indices_are_sorted=True)
        MN = jnp.where(deg[:, None] > 0, MN, 0.0)
        MX = jnp.where(deg[:, None] > 0, MX, 0.0)
        y, stats = _k_post(ns, A, S1, S2, MN, MX, y, stats, p[f"wx{i}"],
                           (p[f"w1_{i}"], p[f"w2_{i}"], p[f"w3_{i}"]),
                           p[f"lw{i}"], p[f"bias{i}"], apply_bn=(i > 0))

    out = _k_pool(y, stats, ns, p["mlp_W1"], p["mlp_b1"],
                  p["mlp_W2"], p["mlp_b2"], p["mlp_W3"], p["mlp_b3"])
    return out[:, :1]


# speedup vs baseline: 28.7581x; 1.8822x over previous
import functools
from functools import partial

import jax
import jax.numpy as jnp
import numpy as np
from jax import lax
from jax.experimental import pallas as pl
from jax.experimental.pallas import tpu as pltpu
from jax.experimental.pallas import tpu_sc as plsc

N = 10000
E = 160000
G = 512
T = 5
F = 75
TF = 375
FO = 15
L = 4
HID = 75
Fp = 128
TFp = 384
BN_ROWS = 1000
NT = N // BN_ROWS
DEG_HIST = [0, 0, 0, 0, 0, 0, 0, 0, 100, 300, 600, 900, 1200, 1400, 1500,
            1400, 1200, 900, 600, 300, 100]
_dh = np.asarray(DEG_HIST, np.float64)
AVG_LOG = float((np.log(np.arange(_dh.shape[0]) + 1.0) * _dh).sum() / _dh.sum())


def _pad2(a, r, c):
    return jnp.pad(a, ((0, r - a.shape[0]), (0, c - a.shape[1])))


def _bn_relu(y, stats):
    m, rstd, g, bb = (stats[0:1, :], stats[1:2, :], stats[2:3, :],
                      stats[3:4, :])
    return jnp.maximum((y - m) * rstd * g + bb, 0.0)


def _ab_body(h_ref, stats_ref, wd_ref, ws_ref, et_ref, a_ref, bx_ref,
             *, apply_bn):
    h = h_ref[...]
    if apply_bn:
        h = _bn_relu(h, stats_ref[...])
    a_ref[...] = jnp.dot(h, wd_ref[...], preferred_element_type=jnp.float32)
    b = jnp.dot(h, ws_ref[...], preferred_element_type=jnp.float32)
    bx_ref[...] = b[None, :, :] + et_ref[0:4, :][:, None, :]


def _k_ab(h, stats, wd, ws, et, apply_bn):
    return pl.pallas_call(
        partial(_ab_body, apply_bn=apply_bn),
        grid=(NT,),
        in_specs=[
            pl.BlockSpec((BN_ROWS, Fp), lambda i: (i, 0)),
            pl.BlockSpec((4, Fp), lambda i: (0, 0)),
            pl.BlockSpec((Fp, TFp), lambda i: (0, 0)),
            pl.BlockSpec((Fp, TFp), lambda i: (0, 0)),
            pl.BlockSpec((8, TFp), lambda i: (0, 0)),
        ],
        out_specs=[
            pl.BlockSpec((BN_ROWS, TFp), lambda i: (i, 0)),
            pl.BlockSpec((4, BN_ROWS, TFp), lambda i: (0, i, 0)),
        ],
        out_shape=[
            jax.ShapeDtypeStruct((N, TFp), jnp.float32),
            jax.ShapeDtypeStruct((4, N, TFp), jnp.float32),
        ],
    )(h, stats, wd, ws, et)


NW = 32
NPW = 320
CB = 128
RPV = 344
SEG = 4 * TFp
FBIG = 1e30
EP = E + 2 * CB
RPP = (NW - 1) * NPW + RPV


def _rp(rp_v, i):
    return rp_v[pl.ds(i, 16)][0]


def _sc_issue(bx_hbm, eidx_hbm, idx_v, rows_v, sg, e0, t, buf):
    pltpu.sync_copy(eidx_hbm.at[pl.ds(e0 + t * CB, CB)], idx_v.at[buf])
    pltpu.async_copy(bx_hbm.at[idx_v.at[buf]], rows_v.at[buf], sg[buf])


@functools.lru_cache(maxsize=1)
def _get_sc_edge():
    mesh = plsc.VectorSubcoreMesh(core_axis_name="c", subcore_axis_name="s")
    return functools.partial(
        pl.kernel,
        out_type=jax.ShapeDtypeStruct((N * SEG,), jnp.float32),
        mesh=mesh,
        scratch_types=[
            pltpu.VMEM((RPV,), jnp.int32),
            pltpu.VMEM((2, CB), jnp.int32),
            pltpu.VMEM((2, CB, TFp), jnp.float32),
            pltpu.VMEM((2 * SEG,), jnp.float32),
            pltpu.VMEM((SEG,), jnp.float32),
            pltpu.SemaphoreType.DMA,
            pltpu.SemaphoreType.DMA,
            pltpu.SemaphoreType.DMA,
            pltpu.SemaphoreType.DMA,
        ],
    )(_sc_edge_body)


def _sc_edge_body(bx_hbm, eidx_hbm, rp_hbm, out_hbm, rp_v, idx_v, rows_v,
                  stage_v, carry_v, sg0, sg1, so0, so1):
    w = lax.axis_index("s") * 2 + lax.axis_index("c")
    n_lo = w * NPW
    n_hi = jnp.minimum(N, n_lo + NPW)
    pltpu.sync_copy(rp_hbm.at[pl.ds(n_lo, RPV)], rp_v)
    e_lo = _rp(rp_v, 0)
    e_hi = _rp(rp_v, n_hi - n_lo)
    e0 = (e_lo // 8) * 8
    nb = jnp.maximum((e_hi - e0 + CB - 1) // CB, 1)
    sg = (sg0, sg1)

    _sc_issue(bx_hbm, eidx_hbm, idx_v, rows_v, sg, e0, 0, 0)

    def _drain_out(slot_sem):
        pltpu.make_async_copy(out_hbm.at[pl.ds(0, SEG)],
                              stage_v.at[pl.ds(0, SEG)], slot_sem).wait()

    def node_body(d_rel, cur_t):
        d = n_lo + d_rel
        active = d < n_hi
        e_s = _rp(rp_v, d_rel)
        e_e = _rp(rp_v, d_rel + 1)
        tb_s = (e_s - e0) // CB
        tb_e = jnp.maximum((e_e - 1 - e0) // CB, tb_s)
        nbd = jnp.where(active, tb_e - tb_s + 1, 0)
        slot = d_rel & 1

        @pl.when(active & (d_rel >= 2) & (slot == 0))
        def _():
            _drain_out(so0)

        @pl.when(active & (d_rel >= 2) & (slot == 1))
        def _():
            _drain_out(so1)

        def bi_body(bi, cur_t):
            tb = jnp.minimum(tb_s + bi, nb - 1)
            need = tb > cur_t

            @pl.when(need & ((tb & 1) == 0))
            def _():
                pltpu.make_async_copy(bx_hbm.at[idx_v.at[0]], rows_v.at[0],
                                      sg0).wait()

            @pl.when(need & ((tb & 1) == 1))
            def _():
                pltpu.make_async_copy(bx_hbm.at[idx_v.at[1]], rows_v.at[1],
                                      sg1).wait()

            @pl.when(need & (tb + 1 < nb) & (((tb + 1) & 1) == 0))
            def _():
                _sc_issue(bx_hbm, eidx_hbm, idx_v, rows_v, sg, e0, tb + 1, 0)

            @pl.when(need & (tb + 1 < nb) & (((tb + 1) & 1) == 1))
            def _():
                _sc_issue(bx_hbm, eidx_hbm, idx_v, rows_v, sg, e0, tb + 1, 1)

            batch_lo = e0 + tb * CB
            bs = jnp.maximum(e_s, batch_lo)
            be = jnp.minimum(e_e, batch_lo + CB)
            kk = jnp.maximum(be - bs, 0)
            lbase = bs - batch_lo
            first = bi == 0
            last = bi == nbd - 1
            buf = tb & 1

            def jbody(j, _):
                col = j * 16
                zero = jnp.zeros((16,), jnp.float32)
                ini_s = jnp.where(first, zero, carry_v[pl.ds(col, 16)])
                ini_q = jnp.where(first, zero, carry_v[pl.ds(TFp + col, 16)])
                ini_n = jnp.where(first, zero + FBIG,
                                  carry_v[pl.ds(2 * TFp + col, 16)])
                ini_x = jnp.where(first, zero - FBIG,
                                  carry_v[pl.ds(3 * TFp + col, 16)])

                def ebody(e, acc):
                    s_, q_, n_, x_ = acc
                    v = rows_v[buf, lbase + e, pl.ds(col, 16)]
                    return (s_ + v, q_ + v * v, jnp.minimum(n_, v),
                            jnp.maximum(x_, v))

                s_, q_, n_, x_ = lax.fori_loop(
                    0, kk, ebody, (ini_s, ini_q, ini_n, ini_x))

                @pl.when(last)
                def _():
                    base = slot * SEG + col
                    stage_v[pl.ds(base, 16)] = s_
                    stage_v[pl.ds(base + TFp, 16)] = q_
                    stage_v[pl.ds(base + 2 * TFp, 16)] = n_
                    stage_v[pl.ds(base + 3 * TFp, 16)] = x_

                @pl.when(~last)
                def _():
                    carry_v[pl.ds(col, 16)] = s_
                    carry_v[pl.ds(TFp + col, 16)] = q_
                    carry_v[pl.ds(2 * TFp + col, 16)] = n_
                    carry_v[pl.ds(3 * TFp + col, 16)] = x_

                return 0

            lax.fori_loop(0, TFp // 16, jbody, 0)
            return jnp.maximum(cur_t, tb)

        cur_t = lax.fori_loop(0, nbd, bi_body, cur_t)

        @pl.when(active & (slot == 0))
        def _():
            pltpu.async_copy(stage_v.at[pl.ds(0, SEG)],
                             out_hbm.at[pl.ds(d * SEG, SEG)], so0)

        @pl.when(active & (slot == 1))
        def _():
            pltpu.async_copy(stage_v.at[pl.ds(SEG, SEG)],
                             out_hbm.at[pl.ds(d * SEG, SEG)], so1)

        return cur_t

    lax.fori_loop(0, NPW, node_body, jnp.int32(-1))

    nw_nodes = n_hi - n_lo

    @pl.when(nw_nodes >= 1)
    def _():
        _drain_out(so0)

    @pl.when(nw_nodes >= 2)
    def _():
        _drain_out(so1)


def _post_body(ns_ref, a_ref, out3_ref, h_ref,
               stats_in_ref, wx_ref, w1_ref, w2_ref, w3_ref, lw_ref,
               bias_ref, y_ref, stats_ref, acc_ref, *, apply_bn):
    i = pl.program_id(0)
    deg = ns_ref[:, 0:1]
    inv = ns_ref[:, 1:2]
    sc1 = ns_ref[:, 2:3]
    sc2 = ns_ref[:, 3:4]
    has = ns_ref[:, 4:5]
    A = a_ref[...]
    S1 = out3_ref[:, 0, :]
    S2 = out3_ref[:, 1, :]
    mean = (deg * A + S1) * inv
    sq = (deg * A * A + 2.0 * A * S1 + S2) * inv
    var = jnp.maximum(sq - mean * mean, 0.0)
    std = jnp.sqrt(var + 1e-5)
    mn = has * (A + out3_ref[:, 2, :])
    mx = has * (A + out3_ref[:, 3, :])

    h = h_ref[...]
    if apply_bn:
        h = _bn_relu(h, stats_in_ref[...])
    f32 = jnp.float32
    y = jnp.dot(h, wx_ref[...], preferred_element_type=f32)
    for k, agg in enumerate((mean, mn, mx, std)):
        y0 = jnp.dot(agg, w1_ref[k], preferred_element_type=f32)
        y1 = jnp.dot(agg, w2_ref[k], preferred_element_type=f32)
        y2 = jnp.dot(agg, w3_ref[k], preferred_element_type=f32)
        y = y + y0 + sc1 * y1 + sc2 * y2
    y = y + bias_ref[0:1, :]
    y = jnp.dot(y, lw_ref[...], preferred_element_type=f32) + bias_ref[1:2, :]
    y_ref[...] = y

    @pl.when(i == 0)
    def _():
        acc_ref[...] = jnp.zeros_like(acc_ref)

    acc_ref[0:1, :] += jnp.sum(y, axis=0, keepdims=True)
    acc_ref[1:2, :] += jnp.sum(y * y, axis=0, keepdims=True)

    @pl.when(i == NT - 1)
    def _():
        m = acc_ref[0:1, :] / float(N)
        v = acc_ref[1:2, :] / float(N) - m * m
        rstd = jax.lax.rsqrt(v + 1e-5)
        stats_ref[...] = jnp.concatenate(
            [m, rstd, bias_ref[2:3, :], bias_ref[3:4, :]], axis=0)


def _k_post(ns, A, out3, h, stats_in, wx, w123, lw, bias, apply_bn):
    w1, w2, w3 = w123
    return pl.pallas_call(
        partial(_post_body, apply_bn=apply_bn),
        grid=(NT,),
        in_specs=[
            pl.BlockSpec((BN_ROWS, 128), lambda i: (i, 0)),
            pl.BlockSpec((BN_ROWS, TFp), lambda i: (i, 0)),
            pl.BlockSpec((BN_ROWS, 4, TFp), lambda i: (i, 0, 0)),
            pl.BlockSpec((BN_ROWS, Fp), lambda i: (i, 0)),
            pl.BlockSpec((4, Fp), lambda i: (0, 0)),
            pl.BlockSpec((Fp, Fp), lambda i: (0, 0)),
            pl.BlockSpec((4, TFp, Fp), lambda i: (0, 0, 0)),
            pl.BlockSpec((4, TFp, Fp), lambda i: (0, 0, 0)),
            pl.BlockSpec((4, TFp, Fp), lambda i: (0, 0, 0)),
            pl.BlockSpec((Fp, Fp), lambda i: (0, 0)),
            pl.BlockSpec((4, Fp), lambda i: (0, 0)),
        ],
        out_specs=[
            pl.BlockSpec((BN_ROWS, Fp), lambda i: (i, 0)),
            pl.BlockSpec((4, Fp), lambda i: (0, 0)),
        ],
        out_shape=[
            jax.ShapeDtypeStruct((N, Fp), jnp.float32),
            jax.ShapeDtypeStruct((4, Fp), jnp.float32),
        ],
        scratch_shapes=[pltpu.VMEM((8, Fp), jnp.float32)],
    )(ns, A, out3, h, stats_in, wx, w1, w2, w3, lw, bias)


def _pool_body(y_ref, stats_ref, ns_ref, w1_ref, b1_ref, w2_ref, b2_ref,
               w3_ref, b3_ref, out_ref, acc_ref):
    i = pl.program_id(0)

    @pl.when(i == 0)
    def _():
        acc_ref[...] = jnp.zeros_like(acc_ref)

    h = _bn_relu(y_ref[...], stats_ref[...])
    bid = ns_ref[:, 5].astype(jnp.int32)
    gids = jax.lax.broadcasted_iota(jnp.int32, (G, BN_ROWS), 0)
    onehot = jnp.where(gids == bid[None, :], 1.0, 0.0)
    acc_ref[...] += jnp.dot(onehot, h, preferred_element_type=jnp.float32)

    @pl.when(i == NT - 1)
    def _():
        f32 = jnp.float32
        z = acc_ref[...]
        z = jnp.maximum(jnp.dot(z, w1_ref[...], preferred_element_type=f32)
                        + b1_ref[0:1, :], 0.0)
        z = jnp.maximum(jnp.dot(z, w2_ref[...], preferred_element_type=f32)
                        + b2_ref[0:1, :], 0.0)
        z = jnp.dot(z, w3_ref[...], preferred_element_type=f32) + b3_ref[0:1, :]
        out_ref[...] = 1.0 / (1.0 + jnp.exp(-z))


def _k_pool(y, stats, ns, w1, b1, w2, b2, w3, b3):
    return pl.pallas_call(
        _pool_body,
        grid=(NT,),
        in_specs=[
            pl.BlockSpec((BN_ROWS, Fp), lambda i: (i, 0)),
            pl.BlockSpec((4, Fp), lambda i: (0, 0)),
            pl.BlockSpec((BN_ROWS, 128), lambda i: (i, 0)),
            pl.BlockSpec((Fp, 128), lambda i: (0, 0)),
            pl.BlockSpec((1, 128), lambda i: (0, 0)),
            pl.BlockSpec((128, 128), lambda i: (0, 0)),
            pl.BlockSpec((1, 128), lambda i: (0, 0)),
            pl.BlockSpec((128, 128), lambda i: (0, 0)),
            pl.BlockSpec((1, 128), lambda i: (0, 0)),
        ],
        out_specs=pl.BlockSpec((G, 128), lambda i: (0, 0)),
        out_shape=jax.ShapeDtypeStruct((G, 128), jnp.float32),
        scratch_shapes=[pltpu.VMEM((G, Fp), jnp.float32)],
    )(y, stats, ns, w1, b1, w2, b2, w3, b3)


def _prep_params(params):
    p = {}
    for i in range(L):
        w = params["pre_W"][i]
        wd = w[:, :F].transpose(1, 0, 2).reshape(F, TF)
        ws = w[:, F:2 * F].transpose(1, 0, 2).reshape(F, TF)
        p[f"wd{i}"] = _pad2(wd, Fp, TFp)
        p[f"ws{i}"] = _pad2(ws, Fp, TFp)
        etab = params["edge_emb"] @ params["enc_W"][i] + params["enc_b"][i]
        ET = (jnp.einsum('af,tfg->atg', etab, w[:, 2 * F:])
              + params["pre_b"][i][None]).reshape(4, TF)
        p[f"et{i}"] = _pad2(ET, 8, TFp)
        pw = params["post_W"][i]
        wx = jnp.zeros((Fp, Fp), jnp.float32)
        for t in range(T):
            wx = wx.at[:F, t * FO:(t + 1) * FO].set(pw[t, :F])
        p[f"wx{i}"] = wx
        for s in range(3):
            blocks = []
            for k in range(4):
                wkt = jnp.zeros((TFp, Fp), jnp.float32)
                for t in range(T):
                    rows = pw[t, F + (4 * s + k) * F: F + (4 * s + k + 1) * F]
                    wkt = wkt.at[t * F:(t + 1) * F,
                                 t * FO:(t + 1) * FO].set(rows)
                blocks.append(wkt)
            p[f"w{s + 1}_{i}"] = jnp.stack(blocks)
        p[f"lw{i}"] = _pad2(params["lin_W"][i], Fp, Fp)
        bias = jnp.zeros((4, Fp), jnp.float32)
        bias = bias.at[0, :T * FO].set(params["post_b"][i].reshape(-1))
        bias = bias.at[1, :HID].set(params["lin_b"][i])
        bias = bias.at[2, :HID].set(params["bn_g"][i])
        bias = bias.at[3, :HID].set(params["bn_b"][i])
        p[f"bias{i}"] = bias
    p["mlp_W1"] = _pad2(params["mlp_W1"], Fp, 128)
    p["mlp_b1"] = _pad2(params["mlp_b1"][None, :], 1, 128)
    p["mlp_W2"] = _pad2(params["mlp_W2"], 128, 128)
    p["mlp_b2"] = _pad2(params["mlp_b2"][None, :], 1, 128)
    p["mlp_W3"] = _pad2(params["mlp_W3"], 128, 128)
    p["mlp_b3"] = _pad2(params["mlp_b3"][None, :], 1, 128)
    return p


def kernel(x, edge_index, edge_attr, batch, params):
    src = edge_index[0]
    dst = edge_index[1]
    perm = jnp.argsort(dst)
    dst_s = dst[perm]
    src_s = src[perm]
    attr_s = edge_attr[perm]
    rowptr = jnp.searchsorted(
        dst_s, jnp.arange(N + 1, dtype=jnp.int32)).astype(jnp.int32)
    deg = (rowptr[1:] - rowptr[:-1]).astype(jnp.float32)
    degc = jnp.maximum(deg, 1.0)
    ld = jnp.log(degc + 1.0)
    ns = jnp.zeros((N, 128), jnp.float32)
    ns = ns.at[:, 0].set(deg)
    ns = ns.at[:, 1].set(1.0 / degc)
    ns = ns.at[:, 2].set(ld / AVG_LOG)
    ns = ns.at[:, 3].set(AVG_LOG / ld)
    ns = ns.at[:, 4].set((deg > 0).astype(jnp.float32))
    ns = ns.at[:, 5].set(batch.astype(jnp.float32))

    eidx = (attr_s * N + src_s).astype(jnp.int32)
    eidx_pad = jnp.zeros((EP,), jnp.int32).at[:E].set(eidx)
    rowptr_pad = jnp.full((RPP,), E, jnp.int32).at[:N + 1].set(rowptr)

    p = _prep_params(params)
    h = _pad2(params["node_emb"], 32, Fp)[x]
    stats = jnp.zeros((4, Fp), jnp.float32)
    y = h

    for i in range(L):
        A, Bx = _k_ab(y, stats, p[f"wd{i}"], p[f"ws{i}"], p[f"et{i}"],
                      apply_bn=(i > 0))
        outf = _get_sc_edge()(Bx.reshape(4 * N, TFp), eidx_pad, rowptr_pad)
        out3 = outf.reshape(N, 4, TFp)
        y, stats = _k_post(ns, A, out3, y, stats, p[f"wx{i}"],
                           (p[f"w1_{i}"], p[f"w2_{i}"], p[f"w3_{i}"]),
                           p[f"lw{i}"], p[f"bias{i}"], apply_bn=(i > 0))

    out = _k_pool(y, stats, ns, p["mlp_W1"], p["mlp_b1"],
                  p["mlp_W2"], p["mlp_b2"], p["mlp_W3"], p["mlp_b3"])
    return out[:, :1]
